# Initial kernel scaffold; baseline (speedup 1.0000x reference)
import jax
import jax.numpy as jnp
from jax.experimental import pallas as pl


def kernel(x_0, adj_0_0, inv_0_0, pos, W1, b1, We, be, Wu, bu):
    raise NotImplementedError("write your pallas kernel here")


# speedup vs baseline: 3.4218x; 3.4218x over previous
import functools

import jax
import jax.numpy as jnp
from jax import lax
from jax.experimental import pallas as pl
from jax.experimental.pallas import tpu as pltpu
from jax.experimental.pallas import tpu_sc as plsc

N = 10000
E = 320000
H = 128
NI = 16

NC = 2
NS = 16
L = 16
NW = NC * NS
EPW = E // NW
CH = 80
NCHUNK = EPW // CH
RB = 624
TAIL = N - NS * RB
KV = H // L

_PREC = lax.Precision.HIGHEST


def _proj_body(x_ref, wa_ref, wb_ref, xa_ref, xb_ref):
    x = x_ref[...]
    xa_ref[...] = jnp.dot(x, wa_ref[...], preferred_element_type=jnp.float32,
                          precision=_PREC)
    xb_ref[...] = jnp.dot(x, wb_ref[...], preferred_element_type=jnp.float32,
                          precision=_PREC)


def _zi_body(inv_ref, wc_ref, b1_ref, zi_ref):
    zi_ref[...] = jnp.dot(inv_ref[...], wc_ref[...],
                          preferred_element_type=jnp.float32,
                          precision=_PREC) + b1_ref[...]


def _update_body(x_ref, acc_ref, wua_ref, wub_ref, bu_ref, out_ref):
    x = x_ref[...]
    aggr = acc_ref[0] + acc_ref[1]
    h = (jnp.dot(x, wua_ref[...], preferred_element_type=jnp.float32,
                 precision=_PREC)
         + jnp.dot(aggr, wub_ref[...], preferred_element_type=jnp.float32,
                   precision=_PREC)
         + bu_ref[...])
    out_ref[...] = x + h


_GDN = lax.GatherDimensionNumbers(offset_dims=(), collapsed_slice_dims=(0,),
                                  start_index_map=(0,))


def _lane_shuffle(v, perm):
    return lax.gather(v, perm[:, None], _GDN, (1,),
                      mode=lax.GatherScatterMode.PROMISE_IN_BOUNDS)


def _all_lanes_sum(v):
    lanes = lax.iota(jnp.int32, L)
    for m in (1, 2, 4, 8):
        v = v + _lane_shuffle(v, lanes ^ m)
    return v


def _sc_edge_body(xa_hbm, xb_hbm, zi_hbm, send_hbm, rec_hbm, we_hbm, be_hbm,
                  out_hbm, sidx, ridx, av, bv, ziv, yv, wev, bev,
                  acc, sem_a, sem_b):
    cid = lax.axis_index("c")
    sid = lax.axis_index("s")
    wid = cid * NS + sid

    def _zero_row(i, carry):
        for k in range(KV):
            yv[i, pl.ds(k * L, L)] = jnp.zeros((L,), jnp.float32)
        return carry

    lax.fori_loop(0, CH, _zero_row, 0)
    base_r = pl.multiple_of(sid * RB, 8)
    n_full = RB // CH
    rem = RB - n_full * CH
    for j in range(n_full):
        pltpu.sync_copy(yv, acc.at[pl.ds(base_r + j * CH, CH)])
    pltpu.sync_copy(yv.at[pl.ds(0, rem)],
                    acc.at[pl.ds(base_r + n_full * CH, rem)])

    @pl.when(sid == 0)
    def _zero_tail():
        pltpu.sync_copy(yv.at[pl.ds(0, TAIL)], acc.at[pl.ds(NS * RB, TAIL)])

    pltpu.sync_copy(we_hbm, wev)
    pltpu.sync_copy(be_hbm, bev)
    we_regs = [wev[pl.ds(k * L, L)] for k in range(KV)]
    be_reg = bev[...]

    plsc.subcore_barrier()

    ebase = wid * EPW

    def _chunk(ci, carry):
        off = pl.multiple_of(ebase + ci * CH, CH)
        pltpu.sync_copy(send_hbm.at[pl.ds(off, CH)], sidx)
        pltpu.sync_copy(rec_hbm.at[pl.ds(off, CH)], ridx)
        cpa = pltpu.async_copy(xa_hbm.at[sidx], av, sem_a)
        cpb = pltpu.async_copy(xb_hbm.at[ridx], bv, sem_b)
        pltpu.sync_copy(zi_hbm.at[pl.ds(off, CH)], ziv)
        cpa.wait()
        cpb.wait()

        def _edge(e, ecarry):
            t = be_reg
            ms = []
            for k in range(KV):
                sl = pl.ds(k * L, L)
                z = av[e, sl] + bv[e, sl] + ziv[e, sl]
                sg = 1.0 / (1.0 + jnp.exp(-z))
                m = z * sg
                ms.append(m)
                t = t + m * we_regs[k]
            wvec = _all_lanes_sum(t)
            w = 1.0 / (1.0 + jnp.exp(-wvec))
            for k in range(KV):
                yv[e, pl.ds(k * L, L)] = ms[k] * w
            return ecarry

        lax.fori_loop(0, CH, _edge, 0)
        pltpu.sync_copy(yv, acc.at[ridx], add=True)
        return carry

    lax.fori_loop(0, NCHUNK, _chunk, 0)

    plsc.subcore_barrier()

    for j in range(n_full + 1):
        nr = CH if j < n_full else rem
        r0 = base_r + j * CH
        pltpu.sync_copy(acc.at[pl.ds(r0, nr)], yv.at[pl.ds(0, nr)])
        pltpu.sync_copy(yv.at[pl.ds(0, nr)], out_hbm.at[cid, pl.ds(r0, nr)])

    @pl.when(sid == 0)
    def _write_tail():
        pltpu.sync_copy(acc.at[pl.ds(NS * RB, TAIL)], yv.at[pl.ds(0, TAIL)])
        pltpu.sync_copy(yv.at[pl.ds(0, TAIL)],
                        out_hbm.at[cid, pl.ds(NS * RB, TAIL)])


_sc_edges = functools.partial(
    pl.kernel,
    out_type=jax.ShapeDtypeStruct((NC, N, H), jnp.float32),
    mesh=plsc.VectorSubcoreMesh(core_axis_name="c", subcore_axis_name="s",
                                num_cores=NC, num_subcores=NS),
    scratch_types=[
        pltpu.VMEM((CH,), jnp.int32),
        pltpu.VMEM((CH,), jnp.int32),
        pltpu.VMEM((CH, H), jnp.float32),
        pltpu.VMEM((CH, H), jnp.float32),
        pltpu.VMEM((CH, H), jnp.float32),
        pltpu.VMEM((CH, H), jnp.float32),
        pltpu.VMEM((H,), jnp.float32),
        pltpu.VMEM((L,), jnp.float32),
        pltpu.VMEM_SHARED((N, H), jnp.float32),
        pltpu.SemaphoreType.DMA,
        pltpu.SemaphoreType.DMA,
    ],
)(_sc_edge_body)


def kernel(x_0, adj_0_0, inv_0_0, pos, W1, b1, We, be, Wu, bu):
    wa = W1[:H]
    wb = W1[H:2 * H]
    wc = W1[2 * H:]

    xa, xb = pl.pallas_call(
        _proj_body,
        out_shape=(jax.ShapeDtypeStruct((N, H), jnp.float32),
                   jax.ShapeDtypeStruct((N, H), jnp.float32)),
    )(x_0, wa, wb)

    BE = 6400
    zi = pl.pallas_call(
        _zi_body,
        grid=(E // BE,),
        in_specs=[pl.BlockSpec((BE, NI), lambda i: (i, 0)),
                  pl.BlockSpec((NI, H), lambda i: (0, 0)),
                  pl.BlockSpec((1, H), lambda i: (0, 0))],
        out_specs=pl.BlockSpec((BE, H), lambda i: (i, 0)),
        out_shape=jax.ShapeDtypeStruct((E, H), jnp.float32),
    )(inv_0_0, wc, b1.reshape(1, H))

    send = adj_0_0[0]
    rec = adj_0_0[1]
    we1 = We[:, 0]
    be16 = jnp.concatenate([be, jnp.zeros((L - 1,), jnp.float32)])

    acc = _sc_edges(xa, xb, zi, send, rec, we1, be16)

    x_new = pl.pallas_call(
        _update_body,
        out_shape=jax.ShapeDtypeStruct((N, H), jnp.float32),
    )(x_0, acc, Wu[:H], Wu[H:], bu.reshape(1, H))

    return (x_new, pos)
